# baseline (speedup 1.0000x reference)
import functools

import jax
import jax.numpy as jnp
from jax import lax
from jax.experimental import pallas as pl
from jax.experimental.pallas import tpu as pltpu
from jax.experimental.pallas import tpu_sc as plsc

_HI = jax.lax.Precision.HIGHEST


def _pack_pairs(x):
    *lead, w = x.shape
    xb = x.astype(jnp.bfloat16).reshape(*lead, w // 32, 2, 16)
    xb = jnp.swapaxes(xb, -1, -2)
    pk = lax.bitcast_convert_type(xb, jnp.float32)
    return pk.reshape(*lead, w // 2)


def _head_body(h_ref, q_ref, ws_ref, wqr_ref, wqrb_ref, o_ref):
    D = h_ref.shape[2]
    b = pl.program_id(0)
    h = h_ref[0]
    hw = jnp.dot(h, ws_ref[...], preferred_element_type=jnp.float32,
                 precision=_HI)
    qw_all = jnp.dot(q_ref[...], wqr_ref[...],
                     preferred_element_type=jnp.float32,
                     precision=_HI) + wqrb_ref[...]
    row = lax.broadcasted_iota(jnp.int32, qw_all.shape, 0)
    qw = jnp.sum(jnp.where(row == b, qw_all, 0.0), axis=0, keepdims=True)
    o_ref[0, :, :D] = h
    o_ref[0, :, D:] = hw + qw


def _rela_body(r_ref, wr_ref, o_ref):
    D = r_ref.shape[1]
    r = r_ref[...]
    o_ref[:, :D] = r
    o_ref[:, D:] = jnp.dot(r, wr_ref[...], preferred_element_type=jnp.float32,
                           precision=_HI)


def _post_body(p0_ref, p1_ref, w1_ref, b1_ref, w2_ref, b2_ref, o_ref):
    D = o_ref.shape[1]
    x0 = p0_ref[...]
    x1 = p1_ref[...]
    agg = x0[:, :D] + x1[:, :D]
    cnt = x0[:, D:D + 1] + x1[:, D:D + 1]
    h1 = jnp.dot(agg, w1_ref[...], preferred_element_type=jnp.float32,
                 precision=_HI) + b1_ref[...]
    sel = jnp.dot(h1, w2_ref[...], preferred_element_type=jnp.float32,
                  precision=_HI) + b2_ref[...]
    sel = jnp.maximum(sel, 0.0)
    o_ref[...] = jnp.where(cnt > 0.0, sel, 0.0)


def _build_sc(NROWS, D, A, R, NC0, NC1):
    HW = (D + A) // 2
    RW = (D + A) // 2
    MW = D + 16
    C = 32
    IDB = 8
    RPT = NROWS // 16
    NG = A // 16
    ND = D // 16

    mesh = plsc.VectorSubcoreMesh(core_axis_name="c", subcore_axis_name="s")

    @functools.partial(
        pl.kernel,
        out_type=jax.ShapeDtypeStruct((2, NROWS, MW), jnp.float32),
        mesh=mesh,
        compiler_params=pltpu.CompilerParams(needs_layout_passes=False,
                                             use_tc_tiling_on_sc=False),
        scratch_types=[
            pltpu.VMEM((3, IDB * C), jnp.int32),
            [pltpu.VMEM((C, HW), jnp.float32)] * 2,
            [pltpu.VMEM((C, RW), jnp.float32)] * 2,
            [pltpu.VMEM((C, MW), jnp.float32)] * 2,
            [pltpu.VMEM((C,), jnp.int32)] * 2,
            pltpu.VMEM((A,), jnp.float32),
            pltpu.VMEM_SHARED((NROWS, MW), jnp.float32),
            [pltpu.SemaphoreType.DMA] * 2,
            [pltpu.SemaphoreType.DMA] * 2,
        ],
    )
    def sc_fn(ids_h, head_h, rela_h, wa_h, out_h,
              ids_v, hb, rb, mb, ob, wa_v, acc, gsem, ssem):
        cid = lax.axis_index("c")
        sid = lax.axis_index("s")
        pltpu.sync_copy(wa_h, wa_v)

        z16 = jnp.zeros((16,), jnp.float32)

        def zrow(i, carry):
            for j in range(MW // 16):
                mb[0][i, pl.ds(j * 16, 16)] = z16
                mb[1][i, pl.ds(j * 16, 16)] = z16
            return carry

        lax.fori_loop(0, C, zrow, 0)
        base = sid * RPT
        pos = 0
        while pos < RPT:
            n = min(C, RPT - pos)
            src = mb[(pos // C) % 2]
            pltpu.sync_copy(src.at[pl.ds(0, n)], acc.at[pl.ds(base + pos, n)])
            pos += n

        plsc.subcore_barrier()

        one16 = jnp.ones((16,), jnp.float32)

        def orow(i, carry):
            mb[0][i, pl.ds(D, 16)] = one16
            mb[1][i, pl.ds(D, 16)] = one16
            return carry

        lax.fori_loop(0, C, orow, 0)

        wa_regs = [wa_v[pl.ds(g * 16, 16)] for g in range(NG)]
        idx15 = jnp.full((16,), 15, jnp.int32)
        nchunk = jnp.where(cid == 0, NC0, NC1)
        wbase = jnp.where(cid == 0, sid * (NC0 * C),
                          16 * (NC0 * C) + sid * (NC1 * C))

        def load_idblock(blk):
            pltpu.sync_copy(
                ids_h.at[:, pl.ds(wbase + blk * (IDB * C), IDB * C)], ids_v)

        SPL = 8

        def issue_gathers(t, buf):
            off = lax.rem(t, IDB) * C
            for q in range(C // SPL):
                pltpu.async_copy(
                    head_h.at[ids_v.at[0, pl.ds(off + q * SPL, SPL)]],
                    hb[buf].at[pl.ds(q * SPL, SPL)], gsem[buf])
                pltpu.async_copy(
                    rela_h.at[ids_v.at[1, pl.ds(off + q * SPL, SPL)]],
                    rb[buf].at[pl.ds(q * SPL, SPL)], gsem[buf])

        def wait_gathers(buf):
            pltpu.make_async_copy(head_h.at[pl.ds(0, C)], hb[buf],
                                  gsem[buf]).wait()
            pltpu.make_async_copy(head_h.at[pl.ds(0, C)], rb[buf],
                                  gsem[buf]).wait()

        def wait_scatter(buf):
            pltpu.make_async_copy(mb[buf], acc.at[ob[buf]], ssem[buf]).wait()

        def compute_chunk(buf):
            hv = hb[buf]
            rv = rb[buf]
            mv = mb[buf]

            @plsc.parallel_loop(0, C, unroll=4)
            def edge(i):
                s = None
                for k in range(A // 32):
                    hp = hv[i, pl.ds(D // 2 + k * 16, 16)]
                    rp = rv[i, pl.ds(D // 2 + k * 16, 16)]
                    ha = plsc.unpack(plsc.bitcast(hp, jnp.bfloat16),
                                     format=plsc.PackFormat.INTERLEAVED)
                    ra = plsc.unpack(plsc.bitcast(rp, jnp.bfloat16),
                                     format=plsc.PackFormat.INTERLEAVED)
                    for h2 in range(2):
                        x = jnp.maximum(ha[h2] + ra[h2], 0.0)
                        x = x * wa_regs[2 * k + h2]
                        s = x if s is None else s + x
                c = plsc.cumsum(s)
                z = lax.gather(
                    c, idx15[:, None],
                    lax.GatherDimensionNumbers(
                        offset_dims=(), collapsed_slice_dims=(0,),
                        start_index_map=(0,)),
                    (1,), mode=lax.GatherScatterMode.PROMISE_IN_BOUNDS)
                alpha = 1.0 / (1.0 + jnp.exp(-z))
                for k in range(D // 32):
                    hp = hv[i, pl.ds(k * 16, 16)]
                    rp = rv[i, pl.ds(k * 16, 16)]
                    hm = plsc.unpack(plsc.bitcast(hp, jnp.bfloat16),
                                     format=plsc.PackFormat.INTERLEAVED)
                    rm = plsc.unpack(plsc.bitcast(rp, jnp.bfloat16),
                                     format=plsc.PackFormat.INTERLEAVED)
                    for h2 in range(2):
                        g = 2 * k + h2
                        mv[i, pl.ds(g * 16, 16)] = (hm[h2] * rm[h2] * alpha)

        def step(t, buf, p):
            wait_gathers(buf)

            @pl.when(p >= 1)
            def _():
                wait_scatter(buf)

            off = lax.rem(t, IDB) * C
            for j in range(C // 16):
                ob[buf][pl.ds(j * 16, 16)] = ids_v[2, pl.ds(off + j * 16, 16)]

            @pl.when(jnp.logical_and(lax.rem(t + 1, IDB) == 0,
                                     t + 1 < nchunk))
            def _():
                load_idblock((t + 1) // IDB)

            @pl.when(t + 1 < nchunk)
            def _():
                issue_gathers(t + 1, 1 - buf)

            compute_chunk(buf)
            pltpu.async_copy(mb[buf], acc.at[ob[buf]], ssem[buf], add=True)

        load_idblock(0)
        issue_gathers(0, 0)

        def pair(p, carry):
            step(2 * p, 0, p)
            step(2 * p + 1, 1, p)
            return carry

        lax.fori_loop(0, nchunk // 2, pair, 0)
        wait_scatter(0)
        wait_scatter(1)
        plsc.subcore_barrier()
        pltpu.sync_copy(acc.at[pl.ds(sid * RPT, RPT)],
                        out_h.at[cid, pl.ds(sid * RPT, RPT)])

    return sc_fn


def kernel(query, q_sub, q_rel, hidden, edges, nodes, rela_embed,
           Ws, Wr, Wqr_W, Wqr_b, Wa, mlp_W1, mlp_b1, mlp_W2, mlp_b2):
    B, N, D = hidden.shape
    A = Ws.shape[1]
    R = rela_embed.shape[0]
    BN = B * N
    E = edges.shape[0]
    W = D + A
    MW = D + 16
    NROWS = -(-(BN + 16) // 128) * 128
    NW = 32
    C = 32
    IDB = 8
    NCHUNK = -(-E // (NW * IDB * C)) * IDB
    TOT = 2 * NCHUNK
    NC0 = 224
    NC1 = TOT - NC0
    E_pad = 16 * TOT * C

    head_tab = pl.pallas_call(
        _head_body,
        grid=(B,),
        in_specs=[
            pl.BlockSpec((1, N, D), lambda b: (b, 0, 0)),
            pl.BlockSpec((B, D), lambda b: (0, 0)),
            pl.BlockSpec((D, A), lambda b: (0, 0)),
            pl.BlockSpec((D, A), lambda b: (0, 0)),
            pl.BlockSpec((1, A), lambda b: (0, 0)),
        ],
        out_specs=pl.BlockSpec((1, N, W), lambda b: (b, 0, 0)),
        out_shape=jax.ShapeDtypeStruct((B, N, W), jnp.float32),
    )(hidden, query, Ws, Wqr_W, Wqr_b.reshape(1, A))
    head_tab = head_tab.reshape(BN, W)

    rela_tab = pl.pallas_call(
        _rela_body,
        out_shape=jax.ShapeDtypeStruct((R, W), jnp.float32),
    )(rela_embed, Wr)

    pad = E_pad - E
    sub_p = jnp.concatenate([edges[:, 1], jnp.zeros((pad,), jnp.int32)])
    rel_p = jnp.concatenate([edges[:, 2], jnp.zeros((pad,), jnp.int32)])
    obj_p = jnp.concatenate([edges[:, 3], jnp.full((pad,), BN, jnp.int32)])
    ids_p = jnp.stack([sub_p, rel_p, obj_p])

    head_pk = jnp.concatenate(
        [_pack_pairs(head_tab[:, :D]), _pack_pairs(head_tab[:, D:])], axis=1)
    rela_pk = jnp.concatenate(
        [_pack_pairs(rela_tab[:, :D]), _pack_pairs(rela_tab[:, D:])], axis=1)

    sc_fn = _build_sc(NROWS, D, A, R, NC0, NC1)
    parts = sc_fn(ids_p, head_pk, rela_pk, Wa.reshape(A))

    p0 = parts[0, :BN]
    p1 = parts[1, :BN]
    GB = 10
    RB = BN // GB
    out = pl.pallas_call(
        _post_body,
        grid=(GB,),
        in_specs=[
            pl.BlockSpec((RB, MW), lambda i: (i, 0)),
            pl.BlockSpec((RB, MW), lambda i: (i, 0)),
            pl.BlockSpec((D, D), lambda i: (0, 0)),
            pl.BlockSpec((1, D), lambda i: (0, 0)),
            pl.BlockSpec((D, D), lambda i: (0, 0)),
            pl.BlockSpec((1, D), lambda i: (0, 0)),
        ],
        out_specs=pl.BlockSpec((RB, D), lambda i: (i, 0)),
        out_shape=jax.ShapeDtypeStruct((BN, D), jnp.float32),
    )(p0, p1, mlp_W1, mlp_b1.reshape(1, D), mlp_W2, mlp_b2.reshape(1, D))
    return out.reshape(B, N, D)


# speedup vs baseline: 1.1889x; 1.1889x over previous
import functools

import jax
import jax.numpy as jnp
from jax import lax
from jax.experimental import pallas as pl
from jax.experimental.pallas import tpu as pltpu
from jax.experimental.pallas import tpu_sc as plsc

_HI = jax.lax.Precision.HIGHEST


def _pack_pairs(x):
    *lead, w = x.shape
    xb = x.astype(jnp.bfloat16).reshape(*lead, w // 32, 2, 16)
    xb = jnp.swapaxes(xb, -1, -2)
    pk = lax.bitcast_convert_type(xb, jnp.float32)
    return pk.reshape(*lead, w // 2)


def _head_body(h_ref, q_ref, ws_ref, wqr_ref, wqrb_ref, o_ref):
    D = h_ref.shape[2]
    b = pl.program_id(0)
    h = h_ref[0]
    hw = jnp.dot(h, ws_ref[...], preferred_element_type=jnp.float32,
                 precision=_HI)
    qw_all = jnp.dot(q_ref[...], wqr_ref[...],
                     preferred_element_type=jnp.float32,
                     precision=_HI) + wqrb_ref[...]
    row = lax.broadcasted_iota(jnp.int32, qw_all.shape, 0)
    qw = jnp.sum(jnp.where(row == b, qw_all, 0.0), axis=0, keepdims=True)
    o_ref[0, :, :D] = h
    o_ref[0, :, D:] = hw + qw


def _rela_body(r_ref, wr_ref, o_ref):
    D = r_ref.shape[1]
    r = r_ref[...]
    o_ref[:, :D] = r
    o_ref[:, D:] = jnp.dot(r, wr_ref[...], preferred_element_type=jnp.float32,
                           precision=_HI)


def _post_body(p0_ref, p1_ref, w1_ref, b1_ref, w2_ref, b2_ref, o_ref):
    D = o_ref.shape[1]
    x0 = p0_ref[...]
    x1 = p1_ref[...]
    agg = x0[:, :D] + x1[:, :D]
    cnt = x0[:, D:D + 1] + x1[:, D:D + 1]
    h1 = jnp.dot(agg, w1_ref[...], preferred_element_type=jnp.float32,
                 precision=_HI) + b1_ref[...]
    sel = jnp.dot(h1, w2_ref[...], preferred_element_type=jnp.float32,
                  precision=_HI) + b2_ref[...]
    sel = jnp.maximum(sel, 0.0)
    o_ref[...] = jnp.where(cnt > 0.0, sel, 0.0)


def _build_sc(NROWS, D, A, R, NC0, NC1):
    HW = (D + A) // 2
    RW = (D + A) // 2
    MW = D + 16
    C = 32
    IDB = 8
    RPT = NROWS // 16
    NG = A // 16
    ND = D // 16

    mesh = plsc.VectorSubcoreMesh(core_axis_name="c", subcore_axis_name="s")

    @functools.partial(
        pl.kernel,
        out_type=jax.ShapeDtypeStruct((2, NROWS, MW), jnp.float32),
        mesh=mesh,
        compiler_params=pltpu.CompilerParams(needs_layout_passes=False,
                                             use_tc_tiling_on_sc=False),
        scratch_types=[
            pltpu.VMEM((3, IDB * C), jnp.int32),
            [pltpu.VMEM((C, HW), jnp.float32)] * 2,
            [pltpu.VMEM((C, RW), jnp.float32)] * 2,
            [pltpu.VMEM((C, MW), jnp.float32)] * 2,
            [pltpu.VMEM((C,), jnp.int32)] * 2,
            pltpu.VMEM((A,), jnp.float32),
            pltpu.VMEM_SHARED((NROWS, MW), jnp.float32),
            [pltpu.SemaphoreType.DMA] * 2,
            [pltpu.SemaphoreType.DMA] * 2,
        ],
    )
    def sc_fn(ids_h, head_h, rela_h, wa_h, out_h,
              ids_v, hb, rb, mb, ob, wa_v, acc, gsem, ssem):
        cid = lax.axis_index("c")
        sid = lax.axis_index("s")
        pltpu.sync_copy(wa_h, wa_v)

        z16 = jnp.zeros((16,), jnp.float32)

        def zrow(i, carry):
            for j in range(MW // 16):
                mb[0][i, pl.ds(j * 16, 16)] = z16
                mb[1][i, pl.ds(j * 16, 16)] = z16
            return carry

        lax.fori_loop(0, C, zrow, 0)
        base = sid * RPT
        pos = 0
        while pos < RPT:
            n = min(C, RPT - pos)
            src = mb[(pos // C) % 2]
            pltpu.sync_copy(src.at[pl.ds(0, n)], acc.at[pl.ds(base + pos, n)])
            pos += n

        plsc.subcore_barrier()

        one16 = jnp.ones((16,), jnp.float32)

        def orow(i, carry):
            mb[0][i, pl.ds(D, 16)] = one16
            mb[1][i, pl.ds(D, 16)] = one16
            return carry

        lax.fori_loop(0, C, orow, 0)

        wa_regs = [wa_v[pl.ds(g * 16, 16)] for g in range(NG)]
        idx15 = jnp.full((16,), 15, jnp.int32)
        nchunk = jnp.where(cid == 0, NC0, NC1)
        wbase = jnp.where(cid == 0, sid * (NC0 * C),
                          16 * (NC0 * C) + sid * (NC1 * C))

        def load_idblock(blk):
            pltpu.sync_copy(
                ids_h.at[:, pl.ds(wbase + blk * (IDB * C), IDB * C)], ids_v)

        SPL = 8

        def issue_gathers(t, buf):
            off = lax.rem(t, IDB) * C
            for q in range(C // SPL):
                pltpu.async_copy(
                    head_h.at[ids_v.at[0, pl.ds(off + q * SPL, SPL)]],
                    hb[buf].at[pl.ds(q * SPL, SPL)], gsem[buf])
                pltpu.async_copy(
                    rela_h.at[ids_v.at[1, pl.ds(off + q * SPL, SPL)]],
                    rb[buf].at[pl.ds(q * SPL, SPL)], gsem[buf])

        def wait_gathers(buf):
            pltpu.make_async_copy(head_h.at[pl.ds(0, C)], hb[buf],
                                  gsem[buf]).wait()
            pltpu.make_async_copy(head_h.at[pl.ds(0, C)], rb[buf],
                                  gsem[buf]).wait()

        def wait_scatter(buf):
            pltpu.make_async_copy(mb[buf], acc.at[ob[buf]], ssem[buf]).wait()

        def compute_chunk(buf):
            hv = hb[buf]
            rv = rb[buf]
            mv = mb[buf]

            @plsc.parallel_loop(0, C, unroll=4)
            def edge(i):
                s = None
                for k in range(A // 32):
                    hp = hv[i, pl.ds(D // 2 + k * 16, 16)]
                    rp = rv[i, pl.ds(D // 2 + k * 16, 16)]
                    ha = plsc.unpack(plsc.bitcast(hp, jnp.bfloat16),
                                     format=plsc.PackFormat.INTERLEAVED)
                    ra = plsc.unpack(plsc.bitcast(rp, jnp.bfloat16),
                                     format=plsc.PackFormat.INTERLEAVED)
                    for h2 in range(2):
                        x = jnp.maximum(ha[h2] + ra[h2], 0.0)
                        x = x * wa_regs[2 * k + h2]
                        s = x if s is None else s + x
                c = plsc.cumsum(s)
                z = lax.gather(
                    c, idx15[:, None],
                    lax.GatherDimensionNumbers(
                        offset_dims=(), collapsed_slice_dims=(0,),
                        start_index_map=(0,)),
                    (1,), mode=lax.GatherScatterMode.PROMISE_IN_BOUNDS)
                alpha = 1.0 / (1.0 + jnp.exp(-z))
                for k in range(D // 32):
                    hp = hv[i, pl.ds(k * 16, 16)]
                    rp = rv[i, pl.ds(k * 16, 16)]
                    hm = plsc.unpack(plsc.bitcast(hp, jnp.bfloat16),
                                     format=plsc.PackFormat.INTERLEAVED)
                    rm = plsc.unpack(plsc.bitcast(rp, jnp.bfloat16),
                                     format=plsc.PackFormat.INTERLEAVED)
                    for h2 in range(2):
                        g = 2 * k + h2
                        mv[i, pl.ds(g * 16, 16)] = (hm[h2] * rm[h2] * alpha)

        def step(t, buf, p):
            wait_gathers(buf)

            @pl.when(p >= 1)
            def _():
                wait_scatter(buf)

            off = lax.rem(t, IDB) * C
            for j in range(C // 16):
                ob[buf][pl.ds(j * 16, 16)] = ids_v[2, pl.ds(off + j * 16, 16)]

            @pl.when(jnp.logical_and(lax.rem(t + 1, IDB) == 0,
                                     t + 1 < nchunk))
            def _():
                load_idblock((t + 1) // IDB)

            @pl.when(t + 1 < nchunk)
            def _():
                issue_gathers(t + 1, 1 - buf)

            compute_chunk(buf)
            pltpu.async_copy(mb[buf], acc.at[ob[buf]], ssem[buf], add=True)

        load_idblock(0)
        issue_gathers(0, 0)

        def pair(p, carry):
            step(2 * p, 0, p)
            step(2 * p + 1, 1, p)
            return carry

        lax.fori_loop(0, nchunk // 2, pair, 0)
        wait_scatter(0)
        wait_scatter(1)
        plsc.subcore_barrier()
        pltpu.sync_copy(acc.at[pl.ds(sid * RPT, RPT)],
                        out_h.at[cid, pl.ds(sid * RPT, RPT)])

    return sc_fn


def kernel(query, q_sub, q_rel, hidden, edges, nodes, rela_embed,
           Ws, Wr, Wqr_W, Wqr_b, Wa, mlp_W1, mlp_b1, mlp_W2, mlp_b2):
    B, N, D = hidden.shape
    A = Ws.shape[1]
    R = rela_embed.shape[0]
    BN = B * N
    E = edges.shape[0]
    W = D + A
    MW = D + 16
    NROWS = -(-(BN + 16) // 128) * 128
    NW = 32
    C = 32
    IDB = 8
    NCHUNK = -(-E // (NW * IDB * C)) * IDB
    TOT = 2 * NCHUNK
    NC0 = 416
    NC1 = TOT - NC0
    E_pad = 16 * TOT * C

    head_tab = pl.pallas_call(
        _head_body,
        grid=(B,),
        in_specs=[
            pl.BlockSpec((1, N, D), lambda b: (b, 0, 0)),
            pl.BlockSpec((B, D), lambda b: (0, 0)),
            pl.BlockSpec((D, A), lambda b: (0, 0)),
            pl.BlockSpec((D, A), lambda b: (0, 0)),
            pl.BlockSpec((1, A), lambda b: (0, 0)),
        ],
        out_specs=pl.BlockSpec((1, N, W), lambda b: (b, 0, 0)),
        out_shape=jax.ShapeDtypeStruct((B, N, W), jnp.float32),
    )(hidden, query, Ws, Wqr_W, Wqr_b.reshape(1, A))
    head_tab = head_tab.reshape(BN, W)

    rela_tab = pl.pallas_call(
        _rela_body,
        out_shape=jax.ShapeDtypeStruct((R, W), jnp.float32),
    )(rela_embed, Wr)

    pad = E_pad - E
    sub_p = jnp.concatenate([edges[:, 1], jnp.zeros((pad,), jnp.int32)])
    rel_p = jnp.concatenate([edges[:, 2], jnp.zeros((pad,), jnp.int32)])
    obj_p = jnp.concatenate([edges[:, 3], jnp.full((pad,), BN, jnp.int32)])
    ids_p = jnp.stack([sub_p, rel_p, obj_p])

    head_pk = jnp.concatenate(
        [_pack_pairs(head_tab[:, :D]), _pack_pairs(head_tab[:, D:])], axis=1)
    rela_pk = jnp.concatenate(
        [_pack_pairs(rela_tab[:, :D]), _pack_pairs(rela_tab[:, D:])], axis=1)

    sc_fn = _build_sc(NROWS, D, A, R, NC0, NC1)
    parts = sc_fn(ids_p, head_pk, rela_pk, Wa.reshape(A))

    p0 = parts[0, :BN]
    p1 = parts[1, :BN]
    GB = 10
    RB = BN // GB
    out = pl.pallas_call(
        _post_body,
        grid=(GB,),
        in_specs=[
            pl.BlockSpec((RB, MW), lambda i: (i, 0)),
            pl.BlockSpec((RB, MW), lambda i: (i, 0)),
            pl.BlockSpec((D, D), lambda i: (0, 0)),
            pl.BlockSpec((1, D), lambda i: (0, 0)),
            pl.BlockSpec((D, D), lambda i: (0, 0)),
            pl.BlockSpec((1, D), lambda i: (0, 0)),
        ],
        out_specs=pl.BlockSpec((RB, D), lambda i: (i, 0)),
        out_shape=jax.ShapeDtypeStruct((BN, D), jnp.float32),
    )(p0, p1, mlp_W1, mlp_b1.reshape(1, D), mlp_W2, mlp_b2.reshape(1, D))
    return out.reshape(B, N, D)


# speedup vs baseline: 1.2168x; 1.0235x over previous
import functools

import jax
import jax.numpy as jnp
from jax import lax
from jax.experimental import pallas as pl
from jax.experimental.pallas import tpu as pltpu
from jax.experimental.pallas import tpu_sc as plsc

_HI = jax.lax.Precision.HIGHEST


def _pack_pairs(x):
    *lead, w = x.shape
    xb = x.astype(jnp.bfloat16).reshape(*lead, w // 32, 2, 16)
    xb = jnp.swapaxes(xb, -1, -2)
    pk = lax.bitcast_convert_type(xb, jnp.float32)
    return pk.reshape(*lead, w // 2)


def _head_body(h_ref, q_ref, ws_ref, wqr_ref, wqrb_ref, o_ref):
    D = h_ref.shape[2]
    b = pl.program_id(0)
    h = h_ref[0]
    hw = jnp.dot(h, ws_ref[...], preferred_element_type=jnp.float32,
                 precision=_HI)
    qw_all = jnp.dot(q_ref[...], wqr_ref[...],
                     preferred_element_type=jnp.float32,
                     precision=_HI) + wqrb_ref[...]
    row = lax.broadcasted_iota(jnp.int32, qw_all.shape, 0)
    qw = jnp.sum(jnp.where(row == b, qw_all, 0.0), axis=0, keepdims=True)
    o_ref[0, :, :D] = h
    o_ref[0, :, D:] = hw + qw


def _rela_body(r_ref, wr_ref, o_ref):
    D = r_ref.shape[1]
    r = r_ref[...]
    o_ref[:, :D] = r
    o_ref[:, D:] = jnp.dot(r, wr_ref[...], preferred_element_type=jnp.float32,
                           precision=_HI)


def _post_body(p0_ref, p1_ref, w1_ref, b1_ref, w2_ref, b2_ref, o_ref):
    D = o_ref.shape[1]
    x0 = p0_ref[...]
    x1 = p1_ref[...]
    agg = x0[:, :D] + x1[:, :D]
    cnt = x0[:, D:D + 1] + x1[:, D:D + 1]
    h1 = jnp.dot(agg, w1_ref[...], preferred_element_type=jnp.float32,
                 precision=_HI) + b1_ref[...]
    sel = jnp.dot(h1, w2_ref[...], preferred_element_type=jnp.float32,
                  precision=_HI) + b2_ref[...]
    sel = jnp.maximum(sel, 0.0)
    o_ref[...] = jnp.where(cnt > 0.0, sel, 0.0)


def _build_sc(NROWS, D, A, R, NC0, NC1):
    HW = (D + A) // 2
    RW = (D + A) // 2
    MW = D + 16
    C = 32
    IDB = 8
    RPT = NROWS // 16
    NG = A // 16
    ND = D // 16

    mesh = plsc.VectorSubcoreMesh(core_axis_name="c", subcore_axis_name="s")

    @functools.partial(
        pl.kernel,
        out_type=jax.ShapeDtypeStruct((2, NROWS, MW), jnp.float32),
        mesh=mesh,
        compiler_params=pltpu.CompilerParams(needs_layout_passes=False,
                                             use_tc_tiling_on_sc=False),
        scratch_types=[
            pltpu.VMEM((3, IDB * C), jnp.int32),
            [pltpu.VMEM((C, HW), jnp.float32)] * 2,
            [pltpu.VMEM((C, RW), jnp.float32)] * 2,
            [pltpu.VMEM((C, MW), jnp.float32)] * 2,
            [pltpu.VMEM((C,), jnp.int32)] * 2,
            pltpu.VMEM((A,), jnp.float32),
            pltpu.VMEM_SHARED((NROWS, MW), jnp.float32),
            [pltpu.SemaphoreType.DMA] * 2,
            [pltpu.SemaphoreType.DMA] * 2,
        ],
    )
    def sc_fn(ids_h, head_h, rela_h, wa_h, out_h,
              ids_v, hb, rb, mb, ob, wa_v, acc, gsem, ssem):
        cid = lax.axis_index("c")
        sid = lax.axis_index("s")
        pltpu.sync_copy(wa_h, wa_v)

        z16 = jnp.zeros((16,), jnp.float32)

        def zrow(i, carry):
            for j in range(MW // 16):
                mb[0][i, pl.ds(j * 16, 16)] = z16
                mb[1][i, pl.ds(j * 16, 16)] = z16
            return carry

        lax.fori_loop(0, C, zrow, 0)
        base = sid * RPT
        pos = 0
        while pos < RPT:
            n = min(C, RPT - pos)
            src = mb[(pos // C) % 2]
            pltpu.sync_copy(src.at[pl.ds(0, n)], acc.at[pl.ds(base + pos, n)])
            pos += n

        plsc.subcore_barrier()

        one16 = jnp.ones((16,), jnp.float32)

        def orow(i, carry):
            mb[0][i, pl.ds(D, 16)] = one16
            mb[1][i, pl.ds(D, 16)] = one16
            return carry

        lax.fori_loop(0, C, orow, 0)

        wa_regs = [wa_v[pl.ds(g * 16, 16)] for g in range(NG)]
        idx15 = jnp.full((16,), 15, jnp.int32)
        nchunk = jnp.where(cid == 0, NC0, NC1)
        wbase = jnp.where(cid == 0, sid * (NC0 * C),
                          16 * (NC0 * C) + sid * (NC1 * C))

        def load_idblock(blk):
            pltpu.sync_copy(
                ids_h.at[:, pl.ds(wbase + blk * (IDB * C), IDB * C)], ids_v)

        SPL = 8

        def issue_gathers(t, buf):
            off = lax.rem(t, IDB) * C
            for q in range(C // SPL):
                pltpu.async_copy(
                    head_h.at[ids_v.at[0, pl.ds(off + q * SPL, SPL)]],
                    hb[buf].at[pl.ds(q * SPL, SPL)], gsem[buf])
                pltpu.async_copy(
                    rela_h.at[ids_v.at[1, pl.ds(off + q * SPL, SPL)]],
                    rb[buf].at[pl.ds(q * SPL, SPL)], gsem[buf])

        def wait_gathers(buf):
            pltpu.make_async_copy(head_h.at[pl.ds(0, C)], hb[buf],
                                  gsem[buf]).wait()
            pltpu.make_async_copy(head_h.at[pl.ds(0, C)], rb[buf],
                                  gsem[buf]).wait()

        def wait_scatter(buf):
            pltpu.make_async_copy(mb[buf], acc.at[ob[buf]], ssem[buf]).wait()

        def compute_chunk(buf):
            hv = hb[buf]
            rv = rb[buf]
            mv = mb[buf]

            @plsc.parallel_loop(0, C, unroll=4)
            def edge(i):
                s = None
                for k in range(A // 32):
                    hp = hv[i, pl.ds(D // 2 + k * 16, 16)]
                    rp = rv[i, pl.ds(D // 2 + k * 16, 16)]
                    ha = plsc.unpack(plsc.bitcast(hp, jnp.bfloat16),
                                     format=plsc.PackFormat.INTERLEAVED)
                    ra = plsc.unpack(plsc.bitcast(rp, jnp.bfloat16),
                                     format=plsc.PackFormat.INTERLEAVED)
                    for h2 in range(2):
                        x = jnp.maximum(ha[h2] + ra[h2], 0.0)
                        x = x * wa_regs[2 * k + h2]
                        s = x if s is None else s + x
                c = plsc.cumsum(s)
                z = lax.gather(
                    c, idx15[:, None],
                    lax.GatherDimensionNumbers(
                        offset_dims=(), collapsed_slice_dims=(0,),
                        start_index_map=(0,)),
                    (1,), mode=lax.GatherScatterMode.PROMISE_IN_BOUNDS)
                alpha = 1.0 / (1.0 + jnp.exp(-z))
                for k in range(D // 32):
                    hp = hv[i, pl.ds(k * 16, 16)]
                    rp = rv[i, pl.ds(k * 16, 16)]
                    hm = plsc.unpack(plsc.bitcast(hp, jnp.bfloat16),
                                     format=plsc.PackFormat.INTERLEAVED)
                    rm = plsc.unpack(plsc.bitcast(rp, jnp.bfloat16),
                                     format=plsc.PackFormat.INTERLEAVED)
                    for h2 in range(2):
                        g = 2 * k + h2
                        mv[i, pl.ds(g * 16, 16)] = (hm[h2] * rm[h2] * alpha)

        def step(t, buf, p):
            wait_gathers(buf)

            @pl.when(p >= 1)
            def _():
                wait_scatter(buf)

            off = lax.rem(t, IDB) * C
            for j in range(C // 16):
                ob[buf][pl.ds(j * 16, 16)] = ids_v[2, pl.ds(off + j * 16, 16)]

            @pl.when(jnp.logical_and(lax.rem(t + 1, IDB) == 0,
                                     t + 1 < nchunk))
            def _():
                load_idblock((t + 1) // IDB)

            @pl.when(t + 1 < nchunk)
            def _():
                issue_gathers(t + 1, 1 - buf)

            compute_chunk(buf)
            pltpu.async_copy(mb[buf], acc.at[ob[buf]], ssem[buf], add=True)

        load_idblock(0)
        issue_gathers(0, 0)

        def pair(p, carry):
            step(2 * p, 0, p)
            step(2 * p + 1, 1, p)
            return carry

        lax.fori_loop(0, nchunk // 2, pair, 0)
        wait_scatter(0)
        wait_scatter(1)
        plsc.subcore_barrier()
        pltpu.sync_copy(acc.at[pl.ds(sid * RPT, RPT)],
                        out_h.at[cid, pl.ds(sid * RPT, RPT)])

    return sc_fn


def kernel(query, q_sub, q_rel, hidden, edges, nodes, rela_embed,
           Ws, Wr, Wqr_W, Wqr_b, Wa, mlp_W1, mlp_b1, mlp_W2, mlp_b2):
    B, N, D = hidden.shape
    A = Ws.shape[1]
    R = rela_embed.shape[0]
    BN = B * N
    E = edges.shape[0]
    W = D + A
    MW = D + 16
    NROWS = -(-(BN + 16) // 128) * 128
    NW = 32
    C = 32
    IDB = 8
    NCHUNK = -(-E // (NW * IDB * C)) * IDB
    TOT = 2 * NCHUNK
    NC0 = 432
    NC1 = TOT - NC0
    E_pad = 16 * TOT * C

    head_tab = pl.pallas_call(
        _head_body,
        grid=(B,),
        in_specs=[
            pl.BlockSpec((1, N, D), lambda b: (b, 0, 0)),
            pl.BlockSpec((B, D), lambda b: (0, 0)),
            pl.BlockSpec((D, A), lambda b: (0, 0)),
            pl.BlockSpec((D, A), lambda b: (0, 0)),
            pl.BlockSpec((1, A), lambda b: (0, 0)),
        ],
        out_specs=pl.BlockSpec((1, N, W), lambda b: (b, 0, 0)),
        out_shape=jax.ShapeDtypeStruct((B, N, W), jnp.float32),
    )(hidden, query, Ws, Wqr_W, Wqr_b.reshape(1, A))
    head_tab = head_tab.reshape(BN, W)

    rela_tab = pl.pallas_call(
        _rela_body,
        out_shape=jax.ShapeDtypeStruct((R, W), jnp.float32),
    )(rela_embed, Wr)

    pad = E_pad - E
    sub_p = jnp.concatenate([edges[:, 1], jnp.zeros((pad,), jnp.int32)])
    rel_p = jnp.concatenate([edges[:, 2], jnp.zeros((pad,), jnp.int32)])
    obj_p = jnp.concatenate([edges[:, 3], jnp.full((pad,), BN, jnp.int32)])
    ids_p = jnp.stack([sub_p, rel_p, obj_p])

    head_pk = jnp.concatenate(
        [_pack_pairs(head_tab[:, :D]), _pack_pairs(head_tab[:, D:])], axis=1)
    rela_pk = jnp.concatenate(
        [_pack_pairs(rela_tab[:, :D]), _pack_pairs(rela_tab[:, D:])], axis=1)

    sc_fn = _build_sc(NROWS, D, A, R, NC0, NC1)
    parts = sc_fn(ids_p, head_pk, rela_pk, Wa.reshape(A))

    p0 = parts[0, :BN]
    p1 = parts[1, :BN]
    GB = 10
    RB = BN // GB
    out = pl.pallas_call(
        _post_body,
        grid=(GB,),
        in_specs=[
            pl.BlockSpec((RB, MW), lambda i: (i, 0)),
            pl.BlockSpec((RB, MW), lambda i: (i, 0)),
            pl.BlockSpec((D, D), lambda i: (0, 0)),
            pl.BlockSpec((1, D), lambda i: (0, 0)),
            pl.BlockSpec((D, D), lambda i: (0, 0)),
            pl.BlockSpec((1, D), lambda i: (0, 0)),
        ],
        out_specs=pl.BlockSpec((RB, D), lambda i: (i, 0)),
        out_shape=jax.ShapeDtypeStruct((BN, D), jnp.float32),
    )(p0, p1, mlp_W1, mlp_b1.reshape(1, D), mlp_W2, mlp_b2.reshape(1, D))
    return out.reshape(B, N, D)


# speedup vs baseline: 1.3434x; 1.1040x over previous
import functools

import jax
import jax.numpy as jnp
from jax import lax
from jax.experimental import pallas as pl
from jax.experimental.pallas import tpu as pltpu
from jax.experimental.pallas import tpu_sc as plsc

_HI = jax.lax.Precision.HIGHEST


def _pack_pairs(x):
    *lead, w = x.shape
    xb = x.astype(jnp.bfloat16).reshape(*lead, w // 32, 2, 16)
    xb = jnp.swapaxes(xb, -1, -2)
    pk = lax.bitcast_convert_type(xb, jnp.float32)
    return pk.reshape(*lead, w // 2)


def _head_body(h_ref, q_ref, ws_ref, wqr_ref, wqrb_ref, r_ref, wr_ref,
               o_ref, ro_ref):
    D = h_ref.shape[2]
    b = pl.program_id(0)
    h = h_ref[0]
    hw = jnp.dot(h, ws_ref[...], preferred_element_type=jnp.float32,
                 precision=_HI)
    qw_all = jnp.dot(q_ref[...], wqr_ref[...],
                     preferred_element_type=jnp.float32,
                     precision=_HI) + wqrb_ref[...]
    row = lax.broadcasted_iota(jnp.int32, qw_all.shape, 0)
    qw = jnp.sum(jnp.where(row == b, qw_all, 0.0), axis=0, keepdims=True)
    o_ref[0, :, :D] = h
    o_ref[0, :, D:] = hw + qw

    @pl.when(b == 0)
    def _():
        r = r_ref[...]
        ro_ref[:, :D] = r
        ro_ref[:, D:] = jnp.dot(r, wr_ref[...],
                                preferred_element_type=jnp.float32,
                                precision=_HI)


def _post_body(p0_ref, p1_ref, w1_ref, b1_ref, w2_ref, b2_ref, o_ref):
    D = o_ref.shape[1]
    x0 = p0_ref[0]
    x1 = p1_ref[0]
    agg = x0[:, :D] + x1[:, :D]
    cnt = x0[:, D:D + 1] + x1[:, D:D + 1]
    h1 = jnp.dot(agg, w1_ref[...], preferred_element_type=jnp.float32,
                 precision=_HI) + b1_ref[...]
    sel = jnp.dot(h1, w2_ref[...], preferred_element_type=jnp.float32,
                  precision=_HI) + b2_ref[...]
    sel = jnp.maximum(sel, 0.0)
    o_ref[...] = jnp.where(cnt > 0.0, sel, 0.0)


def _build_sc(NROWS, D, A, R, NC0, NC1):
    HW = (D + A) // 2
    RW = (D + A) // 2
    MW = D + 16
    C = 32
    IDB = 8
    RPT = NROWS // 16
    NG = A // 16
    ND = D // 16

    mesh = plsc.VectorSubcoreMesh(core_axis_name="c", subcore_axis_name="s")

    @functools.partial(
        pl.kernel,
        out_type=jax.ShapeDtypeStruct((2, NROWS, MW), jnp.float32),
        mesh=mesh,
        compiler_params=pltpu.CompilerParams(needs_layout_passes=False,
                                             use_tc_tiling_on_sc=False),
        scratch_types=[
            pltpu.VMEM((3, IDB * C), jnp.int32),
            [pltpu.VMEM((C, HW), jnp.float32)] * 2,
            [pltpu.VMEM((C, RW), jnp.float32)] * 2,
            [pltpu.VMEM((C, MW), jnp.float32)] * 2,
            [pltpu.VMEM((C,), jnp.int32)] * 2,
            pltpu.VMEM((A,), jnp.float32),
            pltpu.VMEM_SHARED((NROWS, MW), jnp.float32),
            [pltpu.SemaphoreType.DMA] * 2,
            [pltpu.SemaphoreType.DMA] * 2,
        ],
    )
    def sc_fn(ids_h, head_h, rela_h, wa_h, out_h,
              ids_v, hb, rb, mb, ob, wa_v, acc, gsem, ssem):
        cid = lax.axis_index("c")
        sid = lax.axis_index("s")
        pltpu.sync_copy(wa_h, wa_v)

        z16 = jnp.zeros((16,), jnp.float32)

        def zrow(i, carry):
            for j in range(MW // 16):
                mb[0][i, pl.ds(j * 16, 16)] = z16
                mb[1][i, pl.ds(j * 16, 16)] = z16
            return carry

        lax.fori_loop(0, C, zrow, 0)
        base = sid * RPT
        pos = 0
        while pos < RPT:
            n = min(C, RPT - pos)
            src = mb[(pos // C) % 2]
            pltpu.sync_copy(src.at[pl.ds(0, n)], acc.at[pl.ds(base + pos, n)])
            pos += n

        plsc.subcore_barrier()

        one16 = jnp.ones((16,), jnp.float32)

        def orow(i, carry):
            mb[0][i, pl.ds(D, 16)] = one16
            mb[1][i, pl.ds(D, 16)] = one16
            return carry

        lax.fori_loop(0, C, orow, 0)

        wa_regs = [wa_v[pl.ds(g * 16, 16)] for g in range(NG)]
        idx15 = jnp.full((16,), 15, jnp.int32)
        nchunk = jnp.where(cid == 0, NC0, NC1)
        wbase = jnp.where(cid == 0, sid * (NC0 * C),
                          16 * (NC0 * C) + sid * (NC1 * C))

        def load_idblock(blk):
            pltpu.sync_copy(
                ids_h.at[:, pl.ds(wbase + blk * (IDB * C), IDB * C)], ids_v)

        SPL = 8

        def issue_gathers(t, buf):
            off = lax.rem(t, IDB) * C
            for q in range(C // SPL):
                pltpu.async_copy(
                    head_h.at[ids_v.at[0, pl.ds(off + q * SPL, SPL)]],
                    hb[buf].at[pl.ds(q * SPL, SPL)], gsem[buf])
                pltpu.async_copy(
                    rela_h.at[ids_v.at[1, pl.ds(off + q * SPL, SPL)]],
                    rb[buf].at[pl.ds(q * SPL, SPL)], gsem[buf])

        def wait_gathers(buf):
            pltpu.make_async_copy(head_h.at[pl.ds(0, C)], hb[buf],
                                  gsem[buf]).wait()
            pltpu.make_async_copy(head_h.at[pl.ds(0, C)], rb[buf],
                                  gsem[buf]).wait()

        def wait_scatter(buf):
            pltpu.make_async_copy(mb[buf], acc.at[ob[buf]], ssem[buf]).wait()

        def compute_chunk(buf):
            hv = hb[buf]
            rv = rb[buf]
            mv = mb[buf]

            @plsc.parallel_loop(0, C, unroll=4)
            def edge(i):
                s = None
                for k in range(A // 32):
                    hp = hv[i, pl.ds(D // 2 + k * 16, 16)]
                    rp = rv[i, pl.ds(D // 2 + k * 16, 16)]
                    ha = plsc.unpack(plsc.bitcast(hp, jnp.bfloat16),
                                     format=plsc.PackFormat.INTERLEAVED)
                    ra = plsc.unpack(plsc.bitcast(rp, jnp.bfloat16),
                                     format=plsc.PackFormat.INTERLEAVED)
                    for h2 in range(2):
                        x = jnp.maximum(ha[h2] + ra[h2], 0.0)
                        x = x * wa_regs[2 * k + h2]
                        s = x if s is None else s + x
                c = plsc.cumsum(s)
                z = lax.gather(
                    c, idx15[:, None],
                    lax.GatherDimensionNumbers(
                        offset_dims=(), collapsed_slice_dims=(0,),
                        start_index_map=(0,)),
                    (1,), mode=lax.GatherScatterMode.PROMISE_IN_BOUNDS)
                alpha = 1.0 / (1.0 + jnp.exp(-z))
                for k in range(D // 32):
                    hp = hv[i, pl.ds(k * 16, 16)]
                    rp = rv[i, pl.ds(k * 16, 16)]
                    hm = plsc.unpack(plsc.bitcast(hp, jnp.bfloat16),
                                     format=plsc.PackFormat.INTERLEAVED)
                    rm = plsc.unpack(plsc.bitcast(rp, jnp.bfloat16),
                                     format=plsc.PackFormat.INTERLEAVED)
                    for h2 in range(2):
                        g = 2 * k + h2
                        mv[i, pl.ds(g * 16, 16)] = (hm[h2] * rm[h2] * alpha)

        def step(t, buf, p):
            wait_gathers(buf)

            @pl.when(p >= 1)
            def _():
                wait_scatter(buf)

            off = lax.rem(t, IDB) * C
            for j in range(C // 16):
                ob[buf][pl.ds(j * 16, 16)] = ids_v[2, pl.ds(off + j * 16, 16)]

            @pl.when(jnp.logical_and(lax.rem(t + 1, IDB) == 0,
                                     t + 1 < nchunk))
            def _():
                load_idblock((t + 1) // IDB)

            @pl.when(t + 1 < nchunk)
            def _():
                issue_gathers(t + 1, 1 - buf)

            compute_chunk(buf)
            pltpu.async_copy(mb[buf], acc.at[ob[buf]], ssem[buf], add=True)

        load_idblock(0)
        issue_gathers(0, 0)

        def pair(p, carry):
            step(2 * p, 0, p)
            step(2 * p + 1, 1, p)
            return carry

        lax.fori_loop(0, nchunk // 2, pair, 0)
        wait_scatter(0)
        wait_scatter(1)
        plsc.subcore_barrier()
        pltpu.sync_copy(acc.at[pl.ds(sid * RPT, RPT)],
                        out_h.at[cid, pl.ds(sid * RPT, RPT)])

    return sc_fn


def kernel(query, q_sub, q_rel, hidden, edges, nodes, rela_embed,
           Ws, Wr, Wqr_W, Wqr_b, Wa, mlp_W1, mlp_b1, mlp_W2, mlp_b2):
    B, N, D = hidden.shape
    A = Ws.shape[1]
    R = rela_embed.shape[0]
    BN = B * N
    E = edges.shape[0]
    W = D + A
    MW = D + 16
    NROWS = -(-(BN + 16) // 128) * 128
    NW = 32
    C = 32
    IDB = 8
    NCHUNK = -(-E // (NW * IDB * C)) * IDB
    TOT = 2 * NCHUNK
    NC0 = 432
    NC1 = TOT - NC0
    E_pad = 16 * TOT * C

    head_tab, rela_tab = pl.pallas_call(
        _head_body,
        grid=(B,),
        in_specs=[
            pl.BlockSpec((1, N, D), lambda b: (b, 0, 0)),
            pl.BlockSpec((B, D), lambda b: (0, 0)),
            pl.BlockSpec((D, A), lambda b: (0, 0)),
            pl.BlockSpec((D, A), lambda b: (0, 0)),
            pl.BlockSpec((1, A), lambda b: (0, 0)),
            pl.BlockSpec((R, D), lambda b: (0, 0)),
            pl.BlockSpec((D, A), lambda b: (0, 0)),
        ],
        out_specs=[
            pl.BlockSpec((1, N, W), lambda b: (b, 0, 0)),
            pl.BlockSpec((R, W), lambda b: (0, 0)),
        ],
        out_shape=[
            jax.ShapeDtypeStruct((B, N, W), jnp.float32),
            jax.ShapeDtypeStruct((R, W), jnp.float32),
        ],
    )(hidden, query, Ws, Wqr_W, Wqr_b.reshape(1, A), rela_embed, Wr)
    head_tab = head_tab.reshape(BN, W)

    pad = E_pad - E
    sub_p = jnp.concatenate([edges[:, 1], jnp.zeros((pad,), jnp.int32)])
    rel_p = jnp.concatenate([edges[:, 2], jnp.zeros((pad,), jnp.int32)])
    obj_p = jnp.concatenate([edges[:, 3], jnp.full((pad,), BN, jnp.int32)])
    ids_p = jnp.stack([sub_p, rel_p, obj_p])

    head_pk = jnp.concatenate(
        [_pack_pairs(head_tab[:, :D]), _pack_pairs(head_tab[:, D:])], axis=1)
    rela_pk = jnp.concatenate(
        [_pack_pairs(rela_tab[:, :D]), _pack_pairs(rela_tab[:, D:])], axis=1)

    sc_fn = _build_sc(NROWS, D, A, R, NC0, NC1)
    parts = sc_fn(ids_p, head_pk, rela_pk, Wa.reshape(A))

    GB = 10
    RB = BN // GB
    out = pl.pallas_call(
        _post_body,
        grid=(GB,),
        in_specs=[
            pl.BlockSpec((1, RB, MW), lambda i: (0, i, 0)),
            pl.BlockSpec((1, RB, MW), lambda i: (1, i, 0)),
            pl.BlockSpec((D, D), lambda i: (0, 0)),
            pl.BlockSpec((1, D), lambda i: (0, 0)),
            pl.BlockSpec((D, D), lambda i: (0, 0)),
            pl.BlockSpec((1, D), lambda i: (0, 0)),
        ],
        out_specs=pl.BlockSpec((RB, D), lambda i: (i, 0)),
        out_shape=jax.ShapeDtypeStruct((BN, D), jnp.float32),
    )(parts, parts, mlp_W1, mlp_b1.reshape(1, D), mlp_W2,
      mlp_b2.reshape(1, D))
    return out.reshape(B, N, D)
